# Initial kernel scaffold; baseline (speedup 1.0000x reference)
import jax
import jax.numpy as jnp
from jax.experimental import pallas as pl


def kernel(edge_index, h, e, Eh0, Eh1, Eh2, Ee0, Ee1, W_proj, b_proj, W_e1, b_e1, W_e2, b_e2, b_conv, W_ih_gru, W_hh_gru, b_ih_gru, b_hh_gru, W_ih_lstm, W_hh_lstm, b_ih_lstm, b_hh_lstm, W_p1, b_p1, W_p2, b_p2):
    raise NotImplementedError("write your pallas kernel here")


# speedup vs baseline: 12.6632x; 12.6632x over previous
import functools

import jax
import jax.numpy as jnp
from jax import lax
from jax.experimental import pallas as pl
from jax.experimental.pallas import tpu as pltpu
from jax.experimental.pallas import tpu_sc as plsc

N = 10000
E = 320000
D = 16
NK = 64
CHUNK = 128
EP = 327680
NCH = EP // CHUNK
NW = 32
CPW = NCH // NW
NSUB = 16
NP = 10112
RPS = NP // NSUB
T_MP = 3


def _pre_body(h_ref, eh0_ref, eh1_ref, eh2_ref, wp_ref, bp_ref,
              ee0_ref, ee1_ref, we1_ref, be1_ref, we2_ref, be2_ref,
              x_ref, ewt_ref):
    lanes = lax.broadcasted_iota(jnp.int32, (N, 16), 1)
    p0 = jnp.dot(eh0_ref[...], wp_ref[0:8, :], preferred_element_type=jnp.float32)
    p1 = jnp.dot(eh1_ref[...], wp_ref[8:12, :], preferred_element_type=jnp.float32)
    p2 = jnp.dot(eh2_ref[...], wp_ref[12:16, :], preferred_element_type=jnp.float32)
    oh0 = (h_ref[:, 0:1] == lanes).astype(jnp.float32)
    oh1 = (h_ref[:, 1:2] == lanes).astype(jnp.float32)
    oh2 = (h_ref[:, 2:3] == lanes).astype(jnp.float32)
    acc = jnp.dot(oh0, p0, preferred_element_type=jnp.float32)
    acc = acc + jnp.dot(oh1, p1, preferred_element_type=jnp.float32)
    acc = acc + jnp.dot(oh2, p2, preferred_element_type=jnp.float32)
    x_ref[...] = jnp.maximum(acc + bp_ref[...], 0.0)

    kcol = lax.broadcasted_iota(jnp.int32, (NK, 1), 0)
    lanes8 = lax.broadcasted_iota(jnp.int32, (NK, 8), 1)
    ohk0 = ((kcol // 8) == lanes8).astype(jnp.float32)
    ohk1 = ((kcol % 8) == lanes8).astype(jnp.float32)
    g0 = jnp.dot(ee0_ref[...], we1_ref[0:4, :], preferred_element_type=jnp.float32)
    g1 = jnp.dot(ee1_ref[...], we1_ref[4:8, :], preferred_element_type=jnp.float32)
    hmid = jnp.dot(ohk0, g0, preferred_element_type=jnp.float32)
    hmid = hmid + jnp.dot(ohk1, g1, preferred_element_type=jnp.float32)
    hmid = jnp.maximum(hmid + be1_ref[...], 0.0)
    ewt_ref[...] = jnp.dot(hmid, we2_ref[...], preferred_element_type=jnp.float32) + be2_ref[...]


_pre_call = pl.pallas_call(
    _pre_body,
    out_shape=(jax.ShapeDtypeStruct((N, 16), jnp.float32),
               jax.ShapeDtypeStruct((NK, 256), jnp.float32)),
)


_XW_BLK = 2000


def _xw_body(x_ref, wb_ref, o_ref):
    o_ref[...] = jnp.dot(x_ref[...], wb_ref[...], preferred_element_type=jnp.float32)


_xw_call = pl.pallas_call(
    _xw_body,
    grid=(N // _XW_BLK,),
    in_specs=[pl.BlockSpec((_XW_BLK, 16), lambda i: (i, 0)),
              pl.BlockSpec((16, NK * 16), lambda i: (0, 0))],
    out_specs=pl.BlockSpec((_XW_BLK, NK * 16), lambda i: (i, 0)),
    out_shape=jax.ShapeDtypeStruct((N, NK * 16), jnp.float32),
)


def _sc_body(table, gidx, dstp, out, gidx_v, dst_v, rows_v, zbuf, agg_sh, sem):
    c = lax.axis_index("c")
    s = lax.axis_index("s")
    wid = s * 2 + c

    def _z(i, carry):
        zbuf[i, :] = jnp.zeros((16,), jnp.float32)
        return carry
    lax.fori_loop(0, RPS, _z, 0)
    pltpu.sync_copy(zbuf, agg_sh.at[pl.ds(s * RPS, RPS)])

    pltpu.sync_copy(gidx.at[pl.ds(wid * CPW, CPW)], gidx_v)
    pltpu.sync_copy(dstp.at[pl.ds(wid * CPW, CPW)], dst_v)
    plsc.subcore_barrier()

    def _edge_chunk(j, carry):
        pltpu.async_copy(table.at[gidx_v.at[j]], rows_v, sem).wait()
        pltpu.sync_copy(rows_v, agg_sh.at[dst_v.at[j]], add=True)
        return carry
    lax.fori_loop(0, CPW, _edge_chunk, 0)
    plsc.subcore_barrier()

    pltpu.sync_copy(agg_sh.at[pl.ds(s * RPS, RPS)],
                    out.at[c, pl.ds(s * RPS, RPS)])


def _make_sc_call():
  return pl.kernel(
    _sc_body,
    out_type=jax.ShapeDtypeStruct((2, NP, 16), jnp.float32),
    mesh=plsc.VectorSubcoreMesh(core_axis_name="c", subcore_axis_name="s",
                                num_cores=2, num_subcores=NSUB),
    scratch_types=[
        pltpu.VMEM((CPW, CHUNK), jnp.int32),
        pltpu.VMEM((CPW, CHUNK), jnp.int32),
        pltpu.VMEM((CHUNK, 16), jnp.float32),
        pltpu.VMEM((RPS, 16), jnp.float32),
        pltpu.VMEM_SHARED((NP, 16), jnp.float32),
        pltpu.SemaphoreType.DMA,
    ],
    compiler_params=pltpu.CompilerParams(use_tc_tiling_on_sc=False),
  )


def _gru_step(a0, a1, hid, bc, wir, wiz, win, whr, whz, whn,
              bir, biz, bin_, bhr, bhz, bhn):
    x_in = jnp.maximum(a0 + a1 + bc, 0.0)
    dot = functools.partial(jnp.dot, preferred_element_type=jnp.float32)
    r = jax.nn.sigmoid(dot(x_in, wir) + bir + dot(hid, whr) + bhr)
    z = jax.nn.sigmoid(dot(x_in, wiz) + biz + dot(hid, whz) + bhz)
    n = jnp.tanh(dot(x_in, win) + bin_ + r * (dot(hid, whn) + bhn))
    return (1.0 - z) * n + z * hid


def _gruxw_body(a0_ref, a1_ref, hid_ref, bc_ref, wir_ref, wiz_ref, win_ref,
                whr_ref, whz_ref, whn_ref, bir_ref, biz_ref, bin_ref,
                bhr_ref, bhz_ref, bhn_ref, wb_ref, hout_ref, xw_ref):
    hnew = _gru_step(a0_ref[...], a1_ref[...], hid_ref[...], bc_ref[...],
                     wir_ref[...], wiz_ref[...], win_ref[...],
                     whr_ref[...], whz_ref[...], whn_ref[...],
                     bir_ref[...], biz_ref[...], bin_ref[...],
                     bhr_ref[...], bhz_ref[...], bhn_ref[...])
    hout_ref[...] = hnew
    xw_ref[...] = jnp.dot(hnew, wb_ref[...], preferred_element_type=jnp.float32)


_GRU_BLK = 2000
_w16 = pl.BlockSpec((16, 16), lambda i: (0, 0))
_b16 = pl.BlockSpec((1, 16), lambda i: (0, 0))

_gruxw_call = pl.pallas_call(
    _gruxw_body,
    grid=(N // _GRU_BLK,),
    in_specs=[pl.BlockSpec((_GRU_BLK, 16), lambda i: (i, 0)),
              pl.BlockSpec((_GRU_BLK, 16), lambda i: (i, 0)),
              pl.BlockSpec((_GRU_BLK, 16), lambda i: (i, 0)),
              _b16, _w16, _w16, _w16, _w16, _w16, _w16,
              _b16, _b16, _b16, _b16, _b16, _b16,
              pl.BlockSpec((16, NK * 16), lambda i: (0, 0))],
    out_specs=(pl.BlockSpec((_GRU_BLK, 16), lambda i: (i, 0)),
               pl.BlockSpec((_GRU_BLK, NK * 16), lambda i: (i, 0))),
    out_shape=(jax.ShapeDtypeStruct((N, 16), jnp.float32),
               jax.ShapeDtypeStruct((N, NK * 16), jnp.float32)),
)


def _final_body(a0_ref, a1_ref, hid_ref, bc_ref, wir_ref, wiz_ref, win_ref,
                whr_ref, whz_ref, whn_ref, bir_ref, biz_ref, bin_ref,
                bhr_ref, bhz_ref, bhn_ref,
                aq_i_ref, ar_i_ref, hh_i_ref, bl_i_ref,
                aq_f_ref, ar_f_ref, hh_f_ref, bl_f_ref,
                aq_g_ref, ar_g_ref, hh_g_ref, bl_g_ref,
                aq_o_ref, ar_o_ref, hh_o_ref, bl_o_ref,
                wp1q_ref, wp1r_ref, bp1_ref, wp2_ref, bp2_ref, out_ref):
    x = _gru_step(a0_ref[...], a1_ref[...], hid_ref[...], bc_ref[...],
                  wir_ref[...], wiz_ref[...], win_ref[...],
                  whr_ref[...], whz_ref[...], whn_ref[...],
                  bir_ref[...], biz_ref[...], bin_ref[...],
                  bhr_ref[...], bhz_ref[...], bhn_ref[...])
    dot = functools.partial(jnp.dot, preferred_element_type=jnp.float32)
    hc = jnp.zeros((1, 16), jnp.float32)
    cc = jnp.zeros((1, 16), jnp.float32)
    q = jnp.zeros((1, 16), jnp.float32)
    readout = jnp.zeros((1, 16), jnp.float32)
    for _ in range(3):
        i_g = jax.nn.sigmoid(dot(q, aq_i_ref[...]) + dot(readout, ar_i_ref[...])
                             + dot(hc, hh_i_ref[...]) + bl_i_ref[...])
        f_g = jax.nn.sigmoid(dot(q, aq_f_ref[...]) + dot(readout, ar_f_ref[...])
                             + dot(hc, hh_f_ref[...]) + bl_f_ref[...])
        g_g = jnp.tanh(dot(q, aq_g_ref[...]) + dot(readout, ar_g_ref[...])
                       + dot(hc, hh_g_ref[...]) + bl_g_ref[...])
        o_g = jax.nn.sigmoid(dot(q, aq_o_ref[...]) + dot(readout, ar_o_ref[...])
                             + dot(hc, hh_o_ref[...]) + bl_o_ref[...])
        cc = f_g * cc + i_g * g_g
        hc = o_g * jnp.tanh(cc)
        q = hc
        en = jnp.sum(x * q, axis=1, keepdims=True)
        m = jnp.max(en, axis=0, keepdims=True)
        ex = jnp.exp(en - m)
        alpha = ex / jnp.sum(ex, axis=0, keepdims=True)
        readout = jnp.sum(x * alpha, axis=0, keepdims=True)
    hid1 = jnp.maximum(dot(q, wp1q_ref[...]) + dot(readout, wp1r_ref[...])
                       + bp1_ref[...], 0.0)
    out_ref[...] = dot(hid1, wp2_ref[...]) + bp2_ref[...]


_final_call = pl.pallas_call(
    _final_body,
    out_shape=jax.ShapeDtypeStruct((1, 16), jnp.float32),
)


def kernel(edge_index, h, e, Eh0, Eh1, Eh2, Ee0, Ee1, W_proj, b_proj,
           W_e1, b_e1, W_e2, b_e2, b_conv,
           W_ih_gru, W_hh_gru, b_ih_gru, b_hh_gru,
           W_ih_lstm, W_hh_lstm, b_ih_lstm, b_hh_lstm,
           W_p1, b_p1, W_p2, b_p2):
    src = edge_index[0].astype(jnp.int32)
    dst = edge_index[1].astype(jnp.int32)
    eid = e[:, 0].astype(jnp.int32) * 8 + e[:, 1].astype(jnp.int32)
    gidx = src * NK + eid
    gidx_p = jnp.pad(gidx, (0, EP - E)).reshape(NCH, CHUNK)
    dst_p = jnp.pad(dst, (0, EP - E), constant_values=N).reshape(NCH, CHUNK)

    r2 = lambda v: v.reshape(1, -1)
    x, ewt = _pre_call(h.astype(jnp.int32), Eh0, Eh1, Eh2, W_proj, r2(b_proj),
                       Ee0, Ee1, W_e1, r2(b_e1), W_e2, r2(b_e2))
    wbig = ewt.reshape(NK, 16, 16).transpose(1, 0, 2).reshape(16, NK * 16)

    wir, wiz, win = (W_ih_gru[0:16].T, W_ih_gru[16:32].T, W_ih_gru[32:48].T)
    whr, whz, whn = (W_hh_gru[0:16].T, W_hh_gru[16:32].T, W_hh_gru[32:48].T)
    bir, biz, bin_ = r2(b_ih_gru[0:16]), r2(b_ih_gru[16:32]), r2(b_ih_gru[32:48])
    bhr, bhz, bhn = r2(b_hh_gru[0:16]), r2(b_hh_gru[16:32]), r2(b_hh_gru[32:48])
    gru_w = (r2(b_conv), wir, wiz, win, whr, whz, whn,
             bir, biz, bin_, bhr, bhz, bhn)

    bl = b_ih_lstm + b_hh_lstm
    lstm_w = []
    for g in range(4):
        rows = slice(16 * g, 16 * (g + 1))
        lstm_w += [W_ih_lstm[rows, 0:16].T, W_ih_lstm[rows, 16:32].T,
                   W_hh_lstm[rows].T, r2(bl[rows])]

    hidden = x
    xw = _xw_call(x, wbig)
    sc_call = _make_sc_call()
    for t in range(T_MP):
        aggp = sc_call(xw.reshape(N * NK, 16), gidx_p, dst_p)
        if t < T_MP - 1:
            hidden, xw = _gruxw_call(aggp[0], aggp[1], hidden, *gru_w, wbig)
        else:
            out = _final_call(aggp[0, :N], aggp[1, :N], hidden, *gru_w, *lstm_w,
                              W_p1[0:16], W_p1[16:32], r2(b_p1),
                              W_p2, r2(b_p2))
    return out


# speedup vs baseline: 15.5526x; 1.2282x over previous
import functools

import jax
import jax.numpy as jnp
from jax import lax
from jax.experimental import pallas as pl
from jax.experimental.pallas import tpu as pltpu
from jax.experimental.pallas import tpu_sc as plsc

N = 10000
E = 320000
D = 16
NK = 64
CHUNK = 128
EP = 327680
NCH = EP // CHUNK
NW = 32
CPW = NCH // NW
NSUB = 16
NP = 10112
RPS = NP // NSUB
T_MP = 3


def _pre_body(h_ref, eh0_ref, eh1_ref, eh2_ref, wp_ref, bp_ref,
              ee0_ref, ee1_ref, we1_ref, be1_ref, we2_ref, be2_ref,
              x_ref, ewt_ref):
    lanes = lax.broadcasted_iota(jnp.int32, (N, 16), 1)
    p0 = jnp.dot(eh0_ref[...], wp_ref[0:8, :], preferred_element_type=jnp.float32)
    p1 = jnp.dot(eh1_ref[...], wp_ref[8:12, :], preferred_element_type=jnp.float32)
    p2 = jnp.dot(eh2_ref[...], wp_ref[12:16, :], preferred_element_type=jnp.float32)
    oh0 = (h_ref[:, 0:1] == lanes).astype(jnp.float32)
    oh1 = (h_ref[:, 1:2] == lanes).astype(jnp.float32)
    oh2 = (h_ref[:, 2:3] == lanes).astype(jnp.float32)
    acc = jnp.dot(oh0, p0, preferred_element_type=jnp.float32)
    acc = acc + jnp.dot(oh1, p1, preferred_element_type=jnp.float32)
    acc = acc + jnp.dot(oh2, p2, preferred_element_type=jnp.float32)
    x_ref[...] = jnp.maximum(acc + bp_ref[...], 0.0)

    kcol = lax.broadcasted_iota(jnp.int32, (NK, 1), 0)
    lanes8 = lax.broadcasted_iota(jnp.int32, (NK, 8), 1)
    ohk0 = ((kcol // 8) == lanes8).astype(jnp.float32)
    ohk1 = ((kcol % 8) == lanes8).astype(jnp.float32)
    g0 = jnp.dot(ee0_ref[...], we1_ref[0:4, :], preferred_element_type=jnp.float32)
    g1 = jnp.dot(ee1_ref[...], we1_ref[4:8, :], preferred_element_type=jnp.float32)
    hmid = jnp.dot(ohk0, g0, preferred_element_type=jnp.float32)
    hmid = hmid + jnp.dot(ohk1, g1, preferred_element_type=jnp.float32)
    hmid = jnp.maximum(hmid + be1_ref[...], 0.0)
    ewt_ref[...] = jnp.dot(hmid, we2_ref[...], preferred_element_type=jnp.float32) + be2_ref[...]


_pre_call = pl.pallas_call(
    _pre_body,
    out_shape=(jax.ShapeDtypeStruct((N, 16), jnp.float32),
               jax.ShapeDtypeStruct((NK, 256), jnp.float32)),
)


_XW_BLK = 2000


def _xw_body(x_ref, wb_ref, o_ref):
    o_ref[...] = jnp.dot(x_ref[...], wb_ref[...], preferred_element_type=jnp.float32)


_xw_call = pl.pallas_call(
    _xw_body,
    grid=(N // _XW_BLK,),
    in_specs=[pl.BlockSpec((_XW_BLK, 16), lambda i: (i, 0)),
              pl.BlockSpec((16, NK * 16), lambda i: (0, 0))],
    out_specs=pl.BlockSpec((_XW_BLK, NK * 16), lambda i: (i, 0)),
    out_shape=jax.ShapeDtypeStruct((N, NK * 16), jnp.float32),
)


def _sc_body(table, gidx, dstp, out, gidx_v, dst_v, rows_v, zbuf, agg_sh,
             sem_a, sem_b):
    c = lax.axis_index("c")
    s = lax.axis_index("s")
    wid = s * 2 + c

    def _z(i, carry):
        zbuf[i, :] = jnp.zeros((16,), jnp.float32)
        return carry
    lax.fori_loop(0, RPS, _z, 0)
    pltpu.sync_copy(zbuf, agg_sh.at[pl.ds(s * RPS, RPS)])

    pltpu.sync_copy(gidx.at[pl.ds(wid * CPW, CPW)], gidx_v)
    pltpu.sync_copy(dstp.at[pl.ds(wid * CPW, CPW)], dst_v)
    plsc.subcore_barrier()

    def _fire(j, buf, sem):
        pltpu.async_copy(table.at[gidx_v.at[j]], rows_v.at[buf], sem)

    def _drain(j, buf, sem):
        pltpu.make_async_copy(table.at[gidx_v.at[j]], rows_v.at[buf],
                              sem).wait()
        pltpu.sync_copy(rows_v.at[buf], agg_sh.at[dst_v.at[j]], add=True)

    _fire(0, 0, sem_a)
    _fire(1, 1, sem_b)

    def _pair(jj, carry):
        j = jj * 2
        _drain(j, 0, sem_a)
        _fire(j + 2, 0, sem_a)
        _drain(j + 1, 1, sem_b)
        _fire(j + 3, 1, sem_b)
        return carry
    lax.fori_loop(0, CPW // 2 - 1, _pair, 0)
    _drain(CPW - 2, 0, sem_a)
    _drain(CPW - 1, 1, sem_b)
    plsc.subcore_barrier()

    pltpu.sync_copy(agg_sh.at[pl.ds(s * RPS, RPS)],
                    out.at[c, pl.ds(s * RPS, RPS)])


def _make_sc_call():
  return pl.kernel(
    _sc_body,
    out_type=jax.ShapeDtypeStruct((2, NP, 16), jnp.float32),
    mesh=plsc.VectorSubcoreMesh(core_axis_name="c", subcore_axis_name="s",
                                num_cores=2, num_subcores=NSUB),
    scratch_types=[
        pltpu.VMEM((CPW, CHUNK), jnp.int32),
        pltpu.VMEM((CPW, CHUNK), jnp.int32),
        pltpu.VMEM((2, CHUNK, 16), jnp.float32),
        pltpu.VMEM((RPS, 16), jnp.float32),
        pltpu.VMEM_SHARED((NP, 16), jnp.float32),
        pltpu.SemaphoreType.DMA,
        pltpu.SemaphoreType.DMA,
    ],
    compiler_params=pltpu.CompilerParams(use_tc_tiling_on_sc=False),
  )


def _gru_step(a0, a1, hid, bc, wir, wiz, win, whr, whz, whn,
              bir, biz, bin_, bhr, bhz, bhn):
    x_in = jnp.maximum(a0 + a1 + bc, 0.0)
    dot = functools.partial(jnp.dot, preferred_element_type=jnp.float32)
    r = jax.nn.sigmoid(dot(x_in, wir) + bir + dot(hid, whr) + bhr)
    z = jax.nn.sigmoid(dot(x_in, wiz) + biz + dot(hid, whz) + bhz)
    n = jnp.tanh(dot(x_in, win) + bin_ + r * (dot(hid, whn) + bhn))
    return (1.0 - z) * n + z * hid


def _gruxw_body(a0_ref, a1_ref, hid_ref, bc_ref, wir_ref, wiz_ref, win_ref,
                whr_ref, whz_ref, whn_ref, bir_ref, biz_ref, bin_ref,
                bhr_ref, bhz_ref, bhn_ref, wb_ref, hout_ref, xw_ref):
    hnew = _gru_step(a0_ref[...], a1_ref[...], hid_ref[...], bc_ref[...],
                     wir_ref[...], wiz_ref[...], win_ref[...],
                     whr_ref[...], whz_ref[...], whn_ref[...],
                     bir_ref[...], biz_ref[...], bin_ref[...],
                     bhr_ref[...], bhz_ref[...], bhn_ref[...])
    hout_ref[...] = hnew
    xw_ref[...] = jnp.dot(hnew, wb_ref[...], preferred_element_type=jnp.float32)


_GRU_BLK = 2000
_w16 = pl.BlockSpec((16, 16), lambda i: (0, 0))
_b16 = pl.BlockSpec((1, 16), lambda i: (0, 0))

_gruxw_call = pl.pallas_call(
    _gruxw_body,
    grid=(N // _GRU_BLK,),
    in_specs=[pl.BlockSpec((_GRU_BLK, 16), lambda i: (i, 0)),
              pl.BlockSpec((_GRU_BLK, 16), lambda i: (i, 0)),
              pl.BlockSpec((_GRU_BLK, 16), lambda i: (i, 0)),
              _b16, _w16, _w16, _w16, _w16, _w16, _w16,
              _b16, _b16, _b16, _b16, _b16, _b16,
              pl.BlockSpec((16, NK * 16), lambda i: (0, 0))],
    out_specs=(pl.BlockSpec((_GRU_BLK, 16), lambda i: (i, 0)),
               pl.BlockSpec((_GRU_BLK, NK * 16), lambda i: (i, 0))),
    out_shape=(jax.ShapeDtypeStruct((N, 16), jnp.float32),
               jax.ShapeDtypeStruct((N, NK * 16), jnp.float32)),
)


def _final_body(a0_ref, a1_ref, hid_ref, bc_ref, wir_ref, wiz_ref, win_ref,
                whr_ref, whz_ref, whn_ref, bir_ref, biz_ref, bin_ref,
                bhr_ref, bhz_ref, bhn_ref,
                aq_i_ref, ar_i_ref, hh_i_ref, bl_i_ref,
                aq_f_ref, ar_f_ref, hh_f_ref, bl_f_ref,
                aq_g_ref, ar_g_ref, hh_g_ref, bl_g_ref,
                aq_o_ref, ar_o_ref, hh_o_ref, bl_o_ref,
                wp1q_ref, wp1r_ref, bp1_ref, wp2_ref, bp2_ref, out_ref):
    x = _gru_step(a0_ref[...], a1_ref[...], hid_ref[...], bc_ref[...],
                  wir_ref[...], wiz_ref[...], win_ref[...],
                  whr_ref[...], whz_ref[...], whn_ref[...],
                  bir_ref[...], biz_ref[...], bin_ref[...],
                  bhr_ref[...], bhz_ref[...], bhn_ref[...])
    dot = functools.partial(jnp.dot, preferred_element_type=jnp.float32)
    hc = jnp.zeros((1, 16), jnp.float32)
    cc = jnp.zeros((1, 16), jnp.float32)
    q = jnp.zeros((1, 16), jnp.float32)
    readout = jnp.zeros((1, 16), jnp.float32)
    for _ in range(3):
        i_g = jax.nn.sigmoid(dot(q, aq_i_ref[...]) + dot(readout, ar_i_ref[...])
                             + dot(hc, hh_i_ref[...]) + bl_i_ref[...])
        f_g = jax.nn.sigmoid(dot(q, aq_f_ref[...]) + dot(readout, ar_f_ref[...])
                             + dot(hc, hh_f_ref[...]) + bl_f_ref[...])
        g_g = jnp.tanh(dot(q, aq_g_ref[...]) + dot(readout, ar_g_ref[...])
                       + dot(hc, hh_g_ref[...]) + bl_g_ref[...])
        o_g = jax.nn.sigmoid(dot(q, aq_o_ref[...]) + dot(readout, ar_o_ref[...])
                             + dot(hc, hh_o_ref[...]) + bl_o_ref[...])
        cc = f_g * cc + i_g * g_g
        hc = o_g * jnp.tanh(cc)
        q = hc
        en = jnp.sum(x * q, axis=1, keepdims=True)
        m = jnp.max(en, axis=0, keepdims=True)
        ex = jnp.exp(en - m)
        alpha = ex / jnp.sum(ex, axis=0, keepdims=True)
        readout = jnp.sum(x * alpha, axis=0, keepdims=True)
    hid1 = jnp.maximum(dot(q, wp1q_ref[...]) + dot(readout, wp1r_ref[...])
                       + bp1_ref[...], 0.0)
    out_ref[...] = dot(hid1, wp2_ref[...]) + bp2_ref[...]


_final_call = pl.pallas_call(
    _final_body,
    out_shape=jax.ShapeDtypeStruct((1, 16), jnp.float32),
)


def kernel(edge_index, h, e, Eh0, Eh1, Eh2, Ee0, Ee1, W_proj, b_proj,
           W_e1, b_e1, W_e2, b_e2, b_conv,
           W_ih_gru, W_hh_gru, b_ih_gru, b_hh_gru,
           W_ih_lstm, W_hh_lstm, b_ih_lstm, b_hh_lstm,
           W_p1, b_p1, W_p2, b_p2):
    src = edge_index[0].astype(jnp.int32)
    dst = edge_index[1].astype(jnp.int32)
    eid = e[:, 0].astype(jnp.int32) * 8 + e[:, 1].astype(jnp.int32)
    gidx = src * NK + eid
    gidx_p = jnp.pad(gidx, (0, EP - E)).reshape(NCH, CHUNK)
    dst_p = jnp.pad(dst, (0, EP - E), constant_values=N).reshape(NCH, CHUNK)

    r2 = lambda v: v.reshape(1, -1)
    x, ewt = _pre_call(h.astype(jnp.int32), Eh0, Eh1, Eh2, W_proj, r2(b_proj),
                       Ee0, Ee1, W_e1, r2(b_e1), W_e2, r2(b_e2))
    wbig = ewt.reshape(NK, 16, 16).transpose(1, 0, 2).reshape(16, NK * 16)

    wir, wiz, win = (W_ih_gru[0:16].T, W_ih_gru[16:32].T, W_ih_gru[32:48].T)
    whr, whz, whn = (W_hh_gru[0:16].T, W_hh_gru[16:32].T, W_hh_gru[32:48].T)
    bir, biz, bin_ = r2(b_ih_gru[0:16]), r2(b_ih_gru[16:32]), r2(b_ih_gru[32:48])
    bhr, bhz, bhn = r2(b_hh_gru[0:16]), r2(b_hh_gru[16:32]), r2(b_hh_gru[32:48])
    gru_w = (r2(b_conv), wir, wiz, win, whr, whz, whn,
             bir, biz, bin_, bhr, bhz, bhn)

    bl = b_ih_lstm + b_hh_lstm
    lstm_w = []
    for g in range(4):
        rows = slice(16 * g, 16 * (g + 1))
        lstm_w += [W_ih_lstm[rows, 0:16].T, W_ih_lstm[rows, 16:32].T,
                   W_hh_lstm[rows].T, r2(bl[rows])]

    hidden = x
    xw = _xw_call(x, wbig)
    sc_call = _make_sc_call()
    for t in range(T_MP):
        aggp = sc_call(xw.reshape(N * NK, 16), gidx_p, dst_p)
        if t < T_MP - 1:
            hidden, xw = _gruxw_call(aggp[0], aggp[1], hidden, *gru_w, wbig)
        else:
            out = _final_call(aggp[0, :N], aggp[1, :N], hidden, *gru_w, *lstm_w,
                              W_p1[0:16], W_p1[16:32], r2(b_p1),
                              W_p2, r2(b_p2))
    return out


# speedup vs baseline: 17.1257x; 1.1011x over previous
import functools

import jax
import jax.numpy as jnp
from jax import lax
from jax.experimental import pallas as pl
from jax.experimental.pallas import tpu as pltpu
from jax.experimental.pallas import tpu_sc as plsc

N = 10000
E = 320000
D = 16
NK = 64
CHUNK = 128
EP = 327680
NCH = EP // CHUNK
NW = 32
CPW = NCH // NW
NSUB = 16
NP = 10112
RPS = NP // NSUB
T_MP = 3


def _pre_body(h_ref, eh0_ref, eh1_ref, eh2_ref, wp_ref, bp_ref,
              ee0_ref, ee1_ref, we1_ref, be1_ref, we2_ref, be2_ref,
              x_ref, ewt_ref):
    lanes = lax.broadcasted_iota(jnp.int32, (N, 16), 1)
    p0 = jnp.dot(eh0_ref[...], wp_ref[0:8, :], preferred_element_type=jnp.float32)
    p1 = jnp.dot(eh1_ref[...], wp_ref[8:12, :], preferred_element_type=jnp.float32)
    p2 = jnp.dot(eh2_ref[...], wp_ref[12:16, :], preferred_element_type=jnp.float32)
    oh0 = (h_ref[:, 0:1] == lanes).astype(jnp.float32)
    oh1 = (h_ref[:, 1:2] == lanes).astype(jnp.float32)
    oh2 = (h_ref[:, 2:3] == lanes).astype(jnp.float32)
    acc = jnp.dot(oh0, p0, preferred_element_type=jnp.float32)
    acc = acc + jnp.dot(oh1, p1, preferred_element_type=jnp.float32)
    acc = acc + jnp.dot(oh2, p2, preferred_element_type=jnp.float32)
    x_ref[...] = jnp.maximum(acc + bp_ref[...], 0.0)

    kcol = lax.broadcasted_iota(jnp.int32, (NK, 1), 0)
    lanes8 = lax.broadcasted_iota(jnp.int32, (NK, 8), 1)
    ohk0 = ((kcol // 8) == lanes8).astype(jnp.float32)
    ohk1 = ((kcol % 8) == lanes8).astype(jnp.float32)
    g0 = jnp.dot(ee0_ref[...], we1_ref[0:4, :], preferred_element_type=jnp.float32)
    g1 = jnp.dot(ee1_ref[...], we1_ref[4:8, :], preferred_element_type=jnp.float32)
    hmid = jnp.dot(ohk0, g0, preferred_element_type=jnp.float32)
    hmid = hmid + jnp.dot(ohk1, g1, preferred_element_type=jnp.float32)
    hmid = jnp.maximum(hmid + be1_ref[...], 0.0)
    ewt_ref[...] = jnp.dot(hmid, we2_ref[...], preferred_element_type=jnp.float32) + be2_ref[...]


_pre_call = pl.pallas_call(
    _pre_body,
    out_shape=(jax.ShapeDtypeStruct((N, 16), jnp.float32),
               jax.ShapeDtypeStruct((NK, 256), jnp.float32)),
)


_XW_BLK = 2000


def _xw_body(x_ref, wb_ref, o_ref):
    x = x_ref[...]
    for g in range(8):
        o_ref[g] = jnp.dot(x, wb_ref[g], preferred_element_type=jnp.float32)


_xw_call = pl.pallas_call(
    _xw_body,
    grid=(N // _XW_BLK,),
    in_specs=[pl.BlockSpec((_XW_BLK, 16), lambda i: (i, 0)),
              pl.BlockSpec((8, 16, 128), lambda i: (0, 0, 0))],
    out_specs=pl.BlockSpec((8, _XW_BLK, 128), lambda i: (0, i, 0)),
    out_shape=jax.ShapeDtypeStruct((8, N, 128), jnp.float32),
)


def _sc_body(table, gidx, dstp, out, gidx_v, dst_v, rows_v, zbuf, agg_sh,
             sem_a, sem_b):
    c = lax.axis_index("c")
    s = lax.axis_index("s")
    wid = s * 2 + c

    def _z(i, carry):
        zbuf[i, :] = jnp.zeros((16,), jnp.float32)
        return carry
    lax.fori_loop(0, RPS, _z, 0)
    pltpu.sync_copy(zbuf, agg_sh.at[pl.ds(s * RPS, RPS)])

    pltpu.sync_copy(gidx.at[pl.ds(wid * CPW, CPW)], gidx_v)
    pltpu.sync_copy(dstp.at[pl.ds(wid * CPW, CPW)], dst_v)
    plsc.subcore_barrier()

    def _fire(j, buf, sem):
        pltpu.async_copy(table.at[gidx_v.at[j]], rows_v.at[buf], sem)

    def _drain(j, buf, sem):
        pltpu.make_async_copy(table.at[gidx_v.at[j]], rows_v.at[buf],
                              sem).wait()
        pltpu.sync_copy(rows_v.at[buf], agg_sh.at[dst_v.at[j]], add=True)

    _fire(0, 0, sem_a)
    _fire(1, 1, sem_b)

    def _pair(jj, carry):
        j = jj * 2
        _drain(j, 0, sem_a)
        _fire(j + 2, 0, sem_a)
        _drain(j + 1, 1, sem_b)
        _fire(j + 3, 1, sem_b)
        return carry
    lax.fori_loop(0, CPW // 2 - 1, _pair, 0)
    _drain(CPW - 2, 0, sem_a)
    _drain(CPW - 1, 1, sem_b)
    plsc.subcore_barrier()

    pltpu.sync_copy(agg_sh.at[pl.ds(s * RPS, RPS)],
                    out.at[c, pl.ds(s * RPS, RPS)])


def _make_sc_call():
  return pl.kernel(
    _sc_body,
    out_type=jax.ShapeDtypeStruct((2, NP, 16), jnp.float32),
    mesh=plsc.VectorSubcoreMesh(core_axis_name="c", subcore_axis_name="s",
                                num_cores=2, num_subcores=NSUB),
    scratch_types=[
        pltpu.VMEM((CPW, CHUNK), jnp.int32),
        pltpu.VMEM((CPW, CHUNK), jnp.int32),
        pltpu.VMEM((2, CHUNK, 16), jnp.float32),
        pltpu.VMEM((RPS, 16), jnp.float32),
        pltpu.VMEM_SHARED((NP, 16), jnp.float32),
        pltpu.SemaphoreType.DMA,
        pltpu.SemaphoreType.DMA,
    ],
    compiler_params=pltpu.CompilerParams(use_tc_tiling_on_sc=False),
  )


def _gru_step(a0, a1, hid, bc, wir, wiz, win, whr, whz, whn,
              bir, biz, bin_, bhr, bhz, bhn):
    x_in = jnp.maximum(a0 + a1 + bc, 0.0)
    dot = functools.partial(jnp.dot, preferred_element_type=jnp.float32)
    r = jax.nn.sigmoid(dot(x_in, wir) + bir + dot(hid, whr) + bhr)
    z = jax.nn.sigmoid(dot(x_in, wiz) + biz + dot(hid, whz) + bhz)
    n = jnp.tanh(dot(x_in, win) + bin_ + r * (dot(hid, whn) + bhn))
    return (1.0 - z) * n + z * hid


def _gruxw_body(a0_ref, a1_ref, hid_ref, bc_ref, wir_ref, wiz_ref, win_ref,
                whr_ref, whz_ref, whn_ref, bir_ref, biz_ref, bin_ref,
                bhr_ref, bhz_ref, bhn_ref, wb_ref, hout_ref, xw_ref):
    hnew = _gru_step(a0_ref[...], a1_ref[...], hid_ref[...], bc_ref[...],
                     wir_ref[...], wiz_ref[...], win_ref[...],
                     whr_ref[...], whz_ref[...], whn_ref[...],
                     bir_ref[...], biz_ref[...], bin_ref[...],
                     bhr_ref[...], bhz_ref[...], bhn_ref[...])
    hout_ref[...] = hnew
    for g in range(8):
        xw_ref[g] = jnp.dot(hnew, wb_ref[g], preferred_element_type=jnp.float32)


_GRU_BLK = 2000
_w16 = pl.BlockSpec((16, 16), lambda i: (0, 0))
_b16 = pl.BlockSpec((1, 16), lambda i: (0, 0))

_gruxw_call = pl.pallas_call(
    _gruxw_body,
    grid=(N // _GRU_BLK,),
    in_specs=[pl.BlockSpec((_GRU_BLK, 16), lambda i: (i, 0)),
              pl.BlockSpec((_GRU_BLK, 16), lambda i: (i, 0)),
              pl.BlockSpec((_GRU_BLK, 16), lambda i: (i, 0)),
              _b16, _w16, _w16, _w16, _w16, _w16, _w16,
              _b16, _b16, _b16, _b16, _b16, _b16,
              pl.BlockSpec((8, 16, 128), lambda i: (0, 0, 0))],
    out_specs=(pl.BlockSpec((_GRU_BLK, 16), lambda i: (i, 0)),
               pl.BlockSpec((8, _GRU_BLK, 128), lambda i: (0, i, 0))),
    out_shape=(jax.ShapeDtypeStruct((N, 16), jnp.float32),
               jax.ShapeDtypeStruct((8, N, 128), jnp.float32)),
)


def _final_body(a0_ref, a1_ref, hid_ref, bc_ref, wir_ref, wiz_ref, win_ref,
                whr_ref, whz_ref, whn_ref, bir_ref, biz_ref, bin_ref,
                bhr_ref, bhz_ref, bhn_ref,
                aq_i_ref, ar_i_ref, hh_i_ref, bl_i_ref,
                aq_f_ref, ar_f_ref, hh_f_ref, bl_f_ref,
                aq_g_ref, ar_g_ref, hh_g_ref, bl_g_ref,
                aq_o_ref, ar_o_ref, hh_o_ref, bl_o_ref,
                wp1q_ref, wp1r_ref, bp1_ref, wp2_ref, bp2_ref, out_ref):
    x = _gru_step(a0_ref[...], a1_ref[...], hid_ref[...], bc_ref[...],
                  wir_ref[...], wiz_ref[...], win_ref[...],
                  whr_ref[...], whz_ref[...], whn_ref[...],
                  bir_ref[...], biz_ref[...], bin_ref[...],
                  bhr_ref[...], bhz_ref[...], bhn_ref[...])
    dot = functools.partial(jnp.dot, preferred_element_type=jnp.float32)
    hc = jnp.zeros((1, 16), jnp.float32)
    cc = jnp.zeros((1, 16), jnp.float32)
    q = jnp.zeros((1, 16), jnp.float32)
    readout = jnp.zeros((1, 16), jnp.float32)
    for _ in range(3):
        i_g = jax.nn.sigmoid(dot(q, aq_i_ref[...]) + dot(readout, ar_i_ref[...])
                             + dot(hc, hh_i_ref[...]) + bl_i_ref[...])
        f_g = jax.nn.sigmoid(dot(q, aq_f_ref[...]) + dot(readout, ar_f_ref[...])
                             + dot(hc, hh_f_ref[...]) + bl_f_ref[...])
        g_g = jnp.tanh(dot(q, aq_g_ref[...]) + dot(readout, ar_g_ref[...])
                       + dot(hc, hh_g_ref[...]) + bl_g_ref[...])
        o_g = jax.nn.sigmoid(dot(q, aq_o_ref[...]) + dot(readout, ar_o_ref[...])
                             + dot(hc, hh_o_ref[...]) + bl_o_ref[...])
        cc = f_g * cc + i_g * g_g
        hc = o_g * jnp.tanh(cc)
        q = hc
        en = jnp.sum(x * q, axis=1, keepdims=True)
        m = jnp.max(en, axis=0, keepdims=True)
        ex = jnp.exp(en - m)
        alpha = ex / jnp.sum(ex, axis=0, keepdims=True)
        readout = jnp.sum(x * alpha, axis=0, keepdims=True)
    hid1 = jnp.maximum(dot(q, wp1q_ref[...]) + dot(readout, wp1r_ref[...])
                       + bp1_ref[...], 0.0)
    out_ref[...] = dot(hid1, wp2_ref[...]) + bp2_ref[...]


_final_call = pl.pallas_call(
    _final_body,
    out_shape=jax.ShapeDtypeStruct((1, 16), jnp.float32),
)


def kernel(edge_index, h, e, Eh0, Eh1, Eh2, Ee0, Ee1, W_proj, b_proj,
           W_e1, b_e1, W_e2, b_e2, b_conv,
           W_ih_gru, W_hh_gru, b_ih_gru, b_hh_gru,
           W_ih_lstm, W_hh_lstm, b_ih_lstm, b_hh_lstm,
           W_p1, b_p1, W_p2, b_p2):
    src = edge_index[0].astype(jnp.int32)
    dst = edge_index[1].astype(jnp.int32)
    eid = e[:, 0].astype(jnp.int32) * 8 + e[:, 1].astype(jnp.int32)
    gidx = (eid % 8) * (N * 8) + src * 8 + eid // 8
    gidx_p = jnp.pad(gidx, (0, EP - E)).reshape(NCH, CHUNK)
    dst_p = jnp.pad(dst, (0, EP - E), constant_values=N).reshape(NCH, CHUNK)

    r2 = lambda v: v.reshape(1, -1)
    x, ewt = _pre_call(h.astype(jnp.int32), Eh0, Eh1, Eh2, W_proj, r2(b_proj),
                       Ee0, Ee1, W_e1, r2(b_e1), W_e2, r2(b_e2))
    wbig = ewt.reshape(8, 8, 16, 16).transpose(1, 2, 0, 3).reshape(8, 16, 128)

    wir, wiz, win = (W_ih_gru[0:16].T, W_ih_gru[16:32].T, W_ih_gru[32:48].T)
    whr, whz, whn = (W_hh_gru[0:16].T, W_hh_gru[16:32].T, W_hh_gru[32:48].T)
    bir, biz, bin_ = r2(b_ih_gru[0:16]), r2(b_ih_gru[16:32]), r2(b_ih_gru[32:48])
    bhr, bhz, bhn = r2(b_hh_gru[0:16]), r2(b_hh_gru[16:32]), r2(b_hh_gru[32:48])
    gru_w = (r2(b_conv), wir, wiz, win, whr, whz, whn,
             bir, biz, bin_, bhr, bhz, bhn)

    bl = b_ih_lstm + b_hh_lstm
    lstm_w = []
    for g in range(4):
        rows = slice(16 * g, 16 * (g + 1))
        lstm_w += [W_ih_lstm[rows, 0:16].T, W_ih_lstm[rows, 16:32].T,
                   W_hh_lstm[rows].T, r2(bl[rows])]

    hidden = x
    xw = _xw_call(x, wbig)
    sc_call = _make_sc_call()
    for t in range(T_MP):
        aggp = sc_call(xw.reshape(N * NK, 16), gidx_p, dst_p)
        if t < T_MP - 1:
            hidden, xw = _gruxw_call(aggp[0], aggp[1], hidden, *gru_w, wbig)
        else:
            out = _final_call(aggp[0, :N], aggp[1, :N], hidden, *gru_w, *lstm_w,
                              W_p1[0:16], W_p1[16:32], r2(b_p1),
                              W_p2, r2(b_p2))
    return out


# speedup vs baseline: 17.2466x; 1.0071x over previous
import functools

import jax
import jax.numpy as jnp
from jax import lax
from jax.experimental import pallas as pl
from jax.experimental.pallas import tpu as pltpu
from jax.experimental.pallas import tpu_sc as plsc

N = 10000
E = 320000
D = 16
NK = 64
CHUNK = 128
EP = 327680
NCH = EP // CHUNK
NW = 32
CPW = NCH // NW
NSUB = 16
NP = 10112
RPS = NP // NSUB
T_MP = 3


def _pre_body(h_ref, eh0_ref, eh1_ref, eh2_ref, wp_ref, bp_ref,
              ee0_ref, ee1_ref, we1_ref, be1_ref, we2_ref, be2_ref,
              x_ref, ewt_ref):
    lanes = lax.broadcasted_iota(jnp.int32, (N, 16), 1)
    p0 = jnp.dot(eh0_ref[...], wp_ref[0:8, :], preferred_element_type=jnp.float32)
    p1 = jnp.dot(eh1_ref[...], wp_ref[8:12, :], preferred_element_type=jnp.float32)
    p2 = jnp.dot(eh2_ref[...], wp_ref[12:16, :], preferred_element_type=jnp.float32)
    oh0 = (h_ref[:, 0:1] == lanes).astype(jnp.float32)
    oh1 = (h_ref[:, 1:2] == lanes).astype(jnp.float32)
    oh2 = (h_ref[:, 2:3] == lanes).astype(jnp.float32)
    acc = jnp.dot(oh0, p0, preferred_element_type=jnp.float32)
    acc = acc + jnp.dot(oh1, p1, preferred_element_type=jnp.float32)
    acc = acc + jnp.dot(oh2, p2, preferred_element_type=jnp.float32)
    x_ref[...] = jnp.maximum(acc + bp_ref[...], 0.0)

    kcol = lax.broadcasted_iota(jnp.int32, (NK, 1), 0)
    lanes8 = lax.broadcasted_iota(jnp.int32, (NK, 8), 1)
    ohk0 = ((kcol // 8) == lanes8).astype(jnp.float32)
    ohk1 = ((kcol % 8) == lanes8).astype(jnp.float32)
    g0 = jnp.dot(ee0_ref[...], we1_ref[0:4, :], preferred_element_type=jnp.float32)
    g1 = jnp.dot(ee1_ref[...], we1_ref[4:8, :], preferred_element_type=jnp.float32)
    hmid = jnp.dot(ohk0, g0, preferred_element_type=jnp.float32)
    hmid = hmid + jnp.dot(ohk1, g1, preferred_element_type=jnp.float32)
    hmid = jnp.maximum(hmid + be1_ref[...], 0.0)
    ewt_ref[...] = jnp.dot(hmid, we2_ref[...], preferred_element_type=jnp.float32) + be2_ref[...]


_pre_call = pl.pallas_call(
    _pre_body,
    out_shape=(jax.ShapeDtypeStruct((N, 16), jnp.float32),
               jax.ShapeDtypeStruct((NK, 256), jnp.float32)),
)


_XW_BLK = 2000


def _xw_body(x_ref, wb_ref, o_ref):
    x = x_ref[...]
    for g in range(8):
        o_ref[g] = jnp.dot(x, wb_ref[g], preferred_element_type=jnp.float32)


_xw_call = pl.pallas_call(
    _xw_body,
    grid=(N // _XW_BLK,),
    in_specs=[pl.BlockSpec((_XW_BLK, 16), lambda i: (i, 0)),
              pl.BlockSpec((8, 16, 128), lambda i: (0, 0, 0))],
    out_specs=pl.BlockSpec((8, _XW_BLK, 128), lambda i: (0, i, 0)),
    out_shape=jax.ShapeDtypeStruct((8, N, 128), jnp.float32),
)


NBUF = 4


def _sc_body(table, gidx, dstp, out, gidx_v, dst_v, rows_v, zbuf, agg_sh,
             sem_g, sem_s):
    c = lax.axis_index("c")
    s = lax.axis_index("s")
    wid = s * 2 + c

    def _z(i, carry):
        zbuf[i, :] = jnp.zeros((16,), jnp.float32)
        return carry
    lax.fori_loop(0, RPS, _z, 0)
    pltpu.sync_copy(zbuf, agg_sh.at[pl.ds(s * RPS, RPS)])

    pltpu.sync_copy(gidx.at[pl.ds(wid * CPW, CPW)], gidx_v)
    pltpu.sync_copy(dstp.at[pl.ds(wid * CPW, CPW)], dst_v)
    plsc.subcore_barrier()

    def _fire_g(j, b):
        pltpu.async_copy(table.at[gidx_v.at[j]], rows_v.at[b], sem_g.at[b])

    def _wait_g(j, b):
        pltpu.make_async_copy(table.at[gidx_v.at[j]], rows_v.at[b],
                              sem_g.at[b]).wait()

    def _fire_s(j, b):
        pltpu.async_copy(rows_v.at[b], agg_sh.at[dst_v.at[j]], sem_s.at[b],
                         add=True)

    def _wait_s(j, b):
        pltpu.make_async_copy(rows_v.at[b], agg_sh.at[dst_v.at[j]],
                              sem_s.at[b]).wait()

    def _step(j, b, do_wait_s, do_fire_g):
        bg = (b + 2) % 4
        _wait_g(j, b)
        _fire_s(j, b)
        if do_wait_s:
            _wait_s(j - 2, bg)
        if do_fire_g:
            _fire_g(j + 2, bg)

    _fire_g(0, 0)
    _fire_g(1, 1)
    for b in range(4):
        _step(b, b, b >= 2, True)

    def _group(jj, carry):
        j0 = jj * 4
        for b in range(4):
            _step(j0 + b, b, True, True)
        return carry
    lax.fori_loop(1, CPW // 4 - 1, _group, 0)

    for b in range(4):
        _step(CPW - 4 + b, b, True, b < 2)
    _wait_s(CPW - 2, 2)
    _wait_s(CPW - 1, 3)
    plsc.subcore_barrier()

    pltpu.sync_copy(agg_sh.at[pl.ds(s * RPS, RPS)],
                    out.at[c, pl.ds(s * RPS, RPS)])


def _make_sc_call():
  return pl.kernel(
    _sc_body,
    out_type=jax.ShapeDtypeStruct((2, NP, 16), jnp.float32),
    mesh=plsc.VectorSubcoreMesh(core_axis_name="c", subcore_axis_name="s",
                                num_cores=2, num_subcores=NSUB),
    scratch_types=[
        pltpu.VMEM((CPW, CHUNK), jnp.int32),
        pltpu.VMEM((CPW, CHUNK), jnp.int32),
        pltpu.VMEM((NBUF, CHUNK, 16), jnp.float32),
        pltpu.VMEM((RPS, 16), jnp.float32),
        pltpu.VMEM_SHARED((NP, 16), jnp.float32),
        pltpu.SemaphoreType.DMA((NBUF,)),
        pltpu.SemaphoreType.DMA((NBUF,)),
    ],
    compiler_params=pltpu.CompilerParams(use_tc_tiling_on_sc=False),
  )


def _gru_step(a0, a1, hid, bc, wir, wiz, win, whr, whz, whn,
              bir, biz, bin_, bhr, bhz, bhn):
    x_in = jnp.maximum(a0 + a1 + bc, 0.0)
    dot = functools.partial(jnp.dot, preferred_element_type=jnp.float32)
    r = jax.nn.sigmoid(dot(x_in, wir) + bir + dot(hid, whr) + bhr)
    z = jax.nn.sigmoid(dot(x_in, wiz) + biz + dot(hid, whz) + bhz)
    n = jnp.tanh(dot(x_in, win) + bin_ + r * (dot(hid, whn) + bhn))
    return (1.0 - z) * n + z * hid


def _gruxw_body(a0_ref, a1_ref, hid_ref, bc_ref, wir_ref, wiz_ref, win_ref,
                whr_ref, whz_ref, whn_ref, bir_ref, biz_ref, bin_ref,
                bhr_ref, bhz_ref, bhn_ref, wb_ref, hout_ref, xw_ref):
    hnew = _gru_step(a0_ref[...], a1_ref[...], hid_ref[...], bc_ref[...],
                     wir_ref[...], wiz_ref[...], win_ref[...],
                     whr_ref[...], whz_ref[...], whn_ref[...],
                     bir_ref[...], biz_ref[...], bin_ref[...],
                     bhr_ref[...], bhz_ref[...], bhn_ref[...])
    hout_ref[...] = hnew
    for g in range(8):
        xw_ref[g] = jnp.dot(hnew, wb_ref[g], preferred_element_type=jnp.float32)


_GRU_BLK = 2000
_w16 = pl.BlockSpec((16, 16), lambda i: (0, 0))
_b16 = pl.BlockSpec((1, 16), lambda i: (0, 0))

_gruxw_call = pl.pallas_call(
    _gruxw_body,
    grid=(N // _GRU_BLK,),
    in_specs=[pl.BlockSpec((_GRU_BLK, 16), lambda i: (i, 0)),
              pl.BlockSpec((_GRU_BLK, 16), lambda i: (i, 0)),
              pl.BlockSpec((_GRU_BLK, 16), lambda i: (i, 0)),
              _b16, _w16, _w16, _w16, _w16, _w16, _w16,
              _b16, _b16, _b16, _b16, _b16, _b16,
              pl.BlockSpec((8, 16, 128), lambda i: (0, 0, 0))],
    out_specs=(pl.BlockSpec((_GRU_BLK, 16), lambda i: (i, 0)),
               pl.BlockSpec((8, _GRU_BLK, 128), lambda i: (0, i, 0))),
    out_shape=(jax.ShapeDtypeStruct((N, 16), jnp.float32),
               jax.ShapeDtypeStruct((8, N, 128), jnp.float32)),
)


def _final_body(a0_ref, a1_ref, hid_ref, bc_ref, wir_ref, wiz_ref, win_ref,
                whr_ref, whz_ref, whn_ref, bir_ref, biz_ref, bin_ref,
                bhr_ref, bhz_ref, bhn_ref,
                aq_i_ref, ar_i_ref, hh_i_ref, bl_i_ref,
                aq_f_ref, ar_f_ref, hh_f_ref, bl_f_ref,
                aq_g_ref, ar_g_ref, hh_g_ref, bl_g_ref,
                aq_o_ref, ar_o_ref, hh_o_ref, bl_o_ref,
                wp1q_ref, wp1r_ref, bp1_ref, wp2_ref, bp2_ref, out_ref):
    x = _gru_step(a0_ref[...], a1_ref[...], hid_ref[...], bc_ref[...],
                  wir_ref[...], wiz_ref[...], win_ref[...],
                  whr_ref[...], whz_ref[...], whn_ref[...],
                  bir_ref[...], biz_ref[...], bin_ref[...],
                  bhr_ref[...], bhz_ref[...], bhn_ref[...])
    dot = functools.partial(jnp.dot, preferred_element_type=jnp.float32)
    hc = jnp.zeros((1, 16), jnp.float32)
    cc = jnp.zeros((1, 16), jnp.float32)
    q = jnp.zeros((1, 16), jnp.float32)
    readout = jnp.zeros((1, 16), jnp.float32)
    for _ in range(3):
        i_g = jax.nn.sigmoid(dot(q, aq_i_ref[...]) + dot(readout, ar_i_ref[...])
                             + dot(hc, hh_i_ref[...]) + bl_i_ref[...])
        f_g = jax.nn.sigmoid(dot(q, aq_f_ref[...]) + dot(readout, ar_f_ref[...])
                             + dot(hc, hh_f_ref[...]) + bl_f_ref[...])
        g_g = jnp.tanh(dot(q, aq_g_ref[...]) + dot(readout, ar_g_ref[...])
                       + dot(hc, hh_g_ref[...]) + bl_g_ref[...])
        o_g = jax.nn.sigmoid(dot(q, aq_o_ref[...]) + dot(readout, ar_o_ref[...])
                             + dot(hc, hh_o_ref[...]) + bl_o_ref[...])
        cc = f_g * cc + i_g * g_g
        hc = o_g * jnp.tanh(cc)
        q = hc
        en = jnp.sum(x * q, axis=1, keepdims=True)
        m = jnp.max(en, axis=0, keepdims=True)
        ex = jnp.exp(en - m)
        alpha = ex / jnp.sum(ex, axis=0, keepdims=True)
        readout = jnp.sum(x * alpha, axis=0, keepdims=True)
    hid1 = jnp.maximum(dot(q, wp1q_ref[...]) + dot(readout, wp1r_ref[...])
                       + bp1_ref[...], 0.0)
    out_ref[...] = dot(hid1, wp2_ref[...]) + bp2_ref[...]


_final_call = pl.pallas_call(
    _final_body,
    out_shape=jax.ShapeDtypeStruct((1, 16), jnp.float32),
)


def kernel(edge_index, h, e, Eh0, Eh1, Eh2, Ee0, Ee1, W_proj, b_proj,
           W_e1, b_e1, W_e2, b_e2, b_conv,
           W_ih_gru, W_hh_gru, b_ih_gru, b_hh_gru,
           W_ih_lstm, W_hh_lstm, b_ih_lstm, b_hh_lstm,
           W_p1, b_p1, W_p2, b_p2):
    src = edge_index[0].astype(jnp.int32)
    dst = edge_index[1].astype(jnp.int32)
    eid = e[:, 0].astype(jnp.int32) * 8 + e[:, 1].astype(jnp.int32)
    gidx = (eid % 8) * (N * 8) + src * 8 + eid // 8
    gidx_p = jnp.pad(gidx, (0, EP - E)).reshape(NCH, CHUNK)
    dst_p = jnp.pad(dst, (0, EP - E), constant_values=N).reshape(NCH, CHUNK)

    r2 = lambda v: v.reshape(1, -1)
    x, ewt = _pre_call(h.astype(jnp.int32), Eh0, Eh1, Eh2, W_proj, r2(b_proj),
                       Ee0, Ee1, W_e1, r2(b_e1), W_e2, r2(b_e2))
    wbig = ewt.reshape(8, 8, 16, 16).transpose(1, 2, 0, 3).reshape(8, 16, 128)

    wir, wiz, win = (W_ih_gru[0:16].T, W_ih_gru[16:32].T, W_ih_gru[32:48].T)
    whr, whz, whn = (W_hh_gru[0:16].T, W_hh_gru[16:32].T, W_hh_gru[32:48].T)
    bir, biz, bin_ = r2(b_ih_gru[0:16]), r2(b_ih_gru[16:32]), r2(b_ih_gru[32:48])
    bhr, bhz, bhn = r2(b_hh_gru[0:16]), r2(b_hh_gru[16:32]), r2(b_hh_gru[32:48])
    gru_w = (r2(b_conv), wir, wiz, win, whr, whz, whn,
             bir, biz, bin_, bhr, bhz, bhn)

    bl = b_ih_lstm + b_hh_lstm
    lstm_w = []
    for g in range(4):
        rows = slice(16 * g, 16 * (g + 1))
        lstm_w += [W_ih_lstm[rows, 0:16].T, W_ih_lstm[rows, 16:32].T,
                   W_hh_lstm[rows].T, r2(bl[rows])]

    hidden = x
    xw = _xw_call(x, wbig)
    sc_call = _make_sc_call()
    for t in range(T_MP):
        aggp = sc_call(xw.reshape(N * NK, 16), gidx_p, dst_p)
        if t < T_MP - 1:
            hidden, xw = _gruxw_call(aggp[0], aggp[1], hidden, *gru_w, wbig)
        else:
            out = _final_call(aggp[0, :N], aggp[1, :N], hidden, *gru_w, *lstm_w,
                              W_p1[0:16], W_p1[16:32], r2(b_p1),
                              W_p2, r2(b_p2))
    return out


# speedup vs baseline: 17.2734x; 1.0016x over previous
import functools

import jax
import jax.numpy as jnp
from jax import lax
from jax.experimental import pallas as pl
from jax.experimental.pallas import tpu as pltpu
from jax.experimental.pallas import tpu_sc as plsc

N = 10000
E = 320000
D = 16
NK = 64
CHUNK = 128
EP = 327680
NCH = EP // CHUNK
NW = 32
CPW = NCH // NW
NSUB = 16
NP = 10112
RPS = NP // NSUB
T_MP = 3


def _pre_body(h_ref, eh0_ref, eh1_ref, eh2_ref, wp_ref, bp_ref,
              ee0_ref, ee1_ref, we1_ref, be1_ref, we2_ref, be2_ref,
              x_ref, ewt_ref):
    lanes = lax.broadcasted_iota(jnp.int32, (N, 16), 1)
    p0 = jnp.dot(eh0_ref[...], wp_ref[0:8, :], preferred_element_type=jnp.float32)
    p1 = jnp.dot(eh1_ref[...], wp_ref[8:12, :], preferred_element_type=jnp.float32)
    p2 = jnp.dot(eh2_ref[...], wp_ref[12:16, :], preferred_element_type=jnp.float32)
    oh0 = (h_ref[:, 0:1] == lanes).astype(jnp.float32)
    oh1 = (h_ref[:, 1:2] == lanes).astype(jnp.float32)
    oh2 = (h_ref[:, 2:3] == lanes).astype(jnp.float32)
    acc = jnp.dot(oh0, p0, preferred_element_type=jnp.float32)
    acc = acc + jnp.dot(oh1, p1, preferred_element_type=jnp.float32)
    acc = acc + jnp.dot(oh2, p2, preferred_element_type=jnp.float32)
    x_ref[...] = jnp.maximum(acc + bp_ref[...], 0.0)

    kcol = lax.broadcasted_iota(jnp.int32, (NK, 1), 0)
    lanes8 = lax.broadcasted_iota(jnp.int32, (NK, 8), 1)
    ohk0 = ((kcol // 8) == lanes8).astype(jnp.float32)
    ohk1 = ((kcol % 8) == lanes8).astype(jnp.float32)
    g0 = jnp.dot(ee0_ref[...], we1_ref[0:4, :], preferred_element_type=jnp.float32)
    g1 = jnp.dot(ee1_ref[...], we1_ref[4:8, :], preferred_element_type=jnp.float32)
    hmid = jnp.dot(ohk0, g0, preferred_element_type=jnp.float32)
    hmid = hmid + jnp.dot(ohk1, g1, preferred_element_type=jnp.float32)
    hmid = jnp.maximum(hmid + be1_ref[...], 0.0)
    ewt_ref[...] = jnp.dot(hmid, we2_ref[...], preferred_element_type=jnp.float32) + be2_ref[...]


_pre_call = pl.pallas_call(
    _pre_body,
    out_shape=(jax.ShapeDtypeStruct((N, 16), jnp.float32),
               jax.ShapeDtypeStruct((NK, 256), jnp.float32)),
)


_XW_BLK = 2000


def _xw_body(x_ref, wb_ref, o_ref):
    x = x_ref[...]
    for g in range(8):
        o_ref[g] = jnp.dot(x, wb_ref[g], preferred_element_type=jnp.float32)


_xw_call = pl.pallas_call(
    _xw_body,
    grid=(N // _XW_BLK,),
    in_specs=[pl.BlockSpec((_XW_BLK, 16), lambda i: (i, 0)),
              pl.BlockSpec((8, 16, 128), lambda i: (0, 0, 0))],
    out_specs=pl.BlockSpec((8, _XW_BLK, 128), lambda i: (0, i, 0)),
    out_shape=jax.ShapeDtypeStruct((8, N, 128), jnp.float32),
)


NBUF = 8
LOOK = NBUF - 2


def _sc_body(table, gidx, dstp, out, gidx_v, dst_v, rows_v, zbuf, agg_sh,
             sem_g, sem_s):
    c = lax.axis_index("c")
    s = lax.axis_index("s")
    wid = s * 2 + c

    def _z(i, carry):
        zbuf[i, :] = jnp.zeros((16,), jnp.float32)
        return carry
    lax.fori_loop(0, RPS, _z, 0)
    pltpu.sync_copy(zbuf, agg_sh.at[pl.ds(s * RPS, RPS)])

    pltpu.sync_copy(gidx.at[pl.ds(wid * CPW, CPW)], gidx_v)
    pltpu.sync_copy(dstp.at[pl.ds(wid * CPW, CPW)], dst_v)
    plsc.subcore_barrier()

    def _fire_g(j, b):
        pltpu.async_copy(table.at[gidx_v.at[j]], rows_v.at[b], sem_g.at[b])

    def _wait_g(j, b):
        pltpu.make_async_copy(table.at[gidx_v.at[j]], rows_v.at[b],
                              sem_g.at[b]).wait()

    def _fire_s(j, b):
        pltpu.async_copy(rows_v.at[b], agg_sh.at[dst_v.at[j]], sem_s.at[b],
                         add=True)

    def _wait_s(j, b):
        pltpu.make_async_copy(rows_v.at[b], agg_sh.at[dst_v.at[j]],
                              sem_s.at[b]).wait()

    def _step(j, b, do_wait_s, do_fire_g):
        bg = (b + LOOK) % NBUF
        _wait_g(j, b)
        _fire_s(j, b)
        if do_wait_s:
            _wait_s(j - 2, bg)
        if do_fire_g:
            _fire_g(j + LOOK, bg)

    for b in range(LOOK):
        _fire_g(b, b)
    for b in range(NBUF):
        _step(b, b, b >= 2, True)

    def _group(jj, carry):
        j0 = jj * NBUF
        for b in range(NBUF):
            _step(j0 + b, b, True, True)
        return carry
    lax.fori_loop(1, CPW // NBUF - 1, _group, 0)

    for b in range(NBUF):
        _step(CPW - NBUF + b, b, True, b < NBUF - LOOK)
    _wait_s(CPW - 2, (CPW - 2) % NBUF)
    _wait_s(CPW - 1, (CPW - 1) % NBUF)
    plsc.subcore_barrier()

    pltpu.sync_copy(agg_sh.at[pl.ds(s * RPS, RPS)],
                    out.at[c, pl.ds(s * RPS, RPS)])


def _make_sc_call():
  return pl.kernel(
    _sc_body,
    out_type=jax.ShapeDtypeStruct((2, NP, 16), jnp.float32),
    mesh=plsc.VectorSubcoreMesh(core_axis_name="c", subcore_axis_name="s",
                                num_cores=2, num_subcores=NSUB),
    scratch_types=[
        pltpu.VMEM((CPW, CHUNK), jnp.int32),
        pltpu.VMEM((CPW, CHUNK), jnp.int32),
        pltpu.VMEM((NBUF, CHUNK, 16), jnp.float32),
        pltpu.VMEM((RPS, 16), jnp.float32),
        pltpu.VMEM_SHARED((NP, 16), jnp.float32),
        pltpu.SemaphoreType.DMA((NBUF,)),
        pltpu.SemaphoreType.DMA((NBUF,)),
    ],
    compiler_params=pltpu.CompilerParams(use_tc_tiling_on_sc=False),
  )


def _gru_step(a0, a1, hid, bc, wir, wiz, win, whr, whz, whn,
              bir, biz, bin_, bhr, bhz, bhn):
    x_in = jnp.maximum(a0 + a1 + bc, 0.0)
    dot = functools.partial(jnp.dot, preferred_element_type=jnp.float32)
    r = jax.nn.sigmoid(dot(x_in, wir) + bir + dot(hid, whr) + bhr)
    z = jax.nn.sigmoid(dot(x_in, wiz) + biz + dot(hid, whz) + bhz)
    n = jnp.tanh(dot(x_in, win) + bin_ + r * (dot(hid, whn) + bhn))
    return (1.0 - z) * n + z * hid


def _gruxw_body(a0_ref, a1_ref, hid_ref, bc_ref, wir_ref, wiz_ref, win_ref,
                whr_ref, whz_ref, whn_ref, bir_ref, biz_ref, bin_ref,
                bhr_ref, bhz_ref, bhn_ref, wb_ref, hout_ref, xw_ref):
    hnew = _gru_step(a0_ref[...], a1_ref[...], hid_ref[...], bc_ref[...],
                     wir_ref[...], wiz_ref[...], win_ref[...],
                     whr_ref[...], whz_ref[...], whn_ref[...],
                     bir_ref[...], biz_ref[...], bin_ref[...],
                     bhr_ref[...], bhz_ref[...], bhn_ref[...])
    hout_ref[...] = hnew
    for g in range(8):
        xw_ref[g] = jnp.dot(hnew, wb_ref[g], preferred_element_type=jnp.float32)


_GRU_BLK = 2000
_w16 = pl.BlockSpec((16, 16), lambda i: (0, 0))
_b16 = pl.BlockSpec((1, 16), lambda i: (0, 0))

_gruxw_call = pl.pallas_call(
    _gruxw_body,
    grid=(N // _GRU_BLK,),
    in_specs=[pl.BlockSpec((_GRU_BLK, 16), lambda i: (i, 0)),
              pl.BlockSpec((_GRU_BLK, 16), lambda i: (i, 0)),
              pl.BlockSpec((_GRU_BLK, 16), lambda i: (i, 0)),
              _b16, _w16, _w16, _w16, _w16, _w16, _w16,
              _b16, _b16, _b16, _b16, _b16, _b16,
              pl.BlockSpec((8, 16, 128), lambda i: (0, 0, 0))],
    out_specs=(pl.BlockSpec((_GRU_BLK, 16), lambda i: (i, 0)),
               pl.BlockSpec((8, _GRU_BLK, 128), lambda i: (0, i, 0))),
    out_shape=(jax.ShapeDtypeStruct((N, 16), jnp.float32),
               jax.ShapeDtypeStruct((8, N, 128), jnp.float32)),
)


def _final_body(a0_ref, a1_ref, hid_ref, bc_ref, wir_ref, wiz_ref, win_ref,
                whr_ref, whz_ref, whn_ref, bir_ref, biz_ref, bin_ref,
                bhr_ref, bhz_ref, bhn_ref,
                aq_i_ref, ar_i_ref, hh_i_ref, bl_i_ref,
                aq_f_ref, ar_f_ref, hh_f_ref, bl_f_ref,
                aq_g_ref, ar_g_ref, hh_g_ref, bl_g_ref,
                aq_o_ref, ar_o_ref, hh_o_ref, bl_o_ref,
                wp1q_ref, wp1r_ref, bp1_ref, wp2_ref, bp2_ref, out_ref):
    x = _gru_step(a0_ref[...], a1_ref[...], hid_ref[...], bc_ref[...],
                  wir_ref[...], wiz_ref[...], win_ref[...],
                  whr_ref[...], whz_ref[...], whn_ref[...],
                  bir_ref[...], biz_ref[...], bin_ref[...],
                  bhr_ref[...], bhz_ref[...], bhn_ref[...])
    dot = functools.partial(jnp.dot, preferred_element_type=jnp.float32)
    hc = jnp.zeros((1, 16), jnp.float32)
    cc = jnp.zeros((1, 16), jnp.float32)
    q = jnp.zeros((1, 16), jnp.float32)
    readout = jnp.zeros((1, 16), jnp.float32)
    for _ in range(3):
        i_g = jax.nn.sigmoid(dot(q, aq_i_ref[...]) + dot(readout, ar_i_ref[...])
                             + dot(hc, hh_i_ref[...]) + bl_i_ref[...])
        f_g = jax.nn.sigmoid(dot(q, aq_f_ref[...]) + dot(readout, ar_f_ref[...])
                             + dot(hc, hh_f_ref[...]) + bl_f_ref[...])
        g_g = jnp.tanh(dot(q, aq_g_ref[...]) + dot(readout, ar_g_ref[...])
                       + dot(hc, hh_g_ref[...]) + bl_g_ref[...])
        o_g = jax.nn.sigmoid(dot(q, aq_o_ref[...]) + dot(readout, ar_o_ref[...])
                             + dot(hc, hh_o_ref[...]) + bl_o_ref[...])
        cc = f_g * cc + i_g * g_g
        hc = o_g * jnp.tanh(cc)
        q = hc
        en = jnp.sum(x * q, axis=1, keepdims=True)
        m = jnp.max(en, axis=0, keepdims=True)
        ex = jnp.exp(en - m)
        alpha = ex / jnp.sum(ex, axis=0, keepdims=True)
        readout = jnp.sum(x * alpha, axis=0, keepdims=True)
    hid1 = jnp.maximum(dot(q, wp1q_ref[...]) + dot(readout, wp1r_ref[...])
                       + bp1_ref[...], 0.0)
    out_ref[...] = dot(hid1, wp2_ref[...]) + bp2_ref[...]


_final_call = pl.pallas_call(
    _final_body,
    out_shape=jax.ShapeDtypeStruct((1, 16), jnp.float32),
)


def kernel(edge_index, h, e, Eh0, Eh1, Eh2, Ee0, Ee1, W_proj, b_proj,
           W_e1, b_e1, W_e2, b_e2, b_conv,
           W_ih_gru, W_hh_gru, b_ih_gru, b_hh_gru,
           W_ih_lstm, W_hh_lstm, b_ih_lstm, b_hh_lstm,
           W_p1, b_p1, W_p2, b_p2):
    src = edge_index[0].astype(jnp.int32)
    dst = edge_index[1].astype(jnp.int32)
    eid = e[:, 0].astype(jnp.int32) * 8 + e[:, 1].astype(jnp.int32)
    gidx = (eid % 8) * (N * 8) + src * 8 + eid // 8
    gidx_p = jnp.pad(gidx, (0, EP - E)).reshape(NCH, CHUNK)
    dst_p = jnp.pad(dst, (0, EP - E), constant_values=N).reshape(NCH, CHUNK)

    r2 = lambda v: v.reshape(1, -1)
    x, ewt = _pre_call(h.astype(jnp.int32), Eh0, Eh1, Eh2, W_proj, r2(b_proj),
                       Ee0, Ee1, W_e1, r2(b_e1), W_e2, r2(b_e2))
    wbig = ewt.reshape(8, 8, 16, 16).transpose(1, 2, 0, 3).reshape(8, 16, 128)

    wir, wiz, win = (W_ih_gru[0:16].T, W_ih_gru[16:32].T, W_ih_gru[32:48].T)
    whr, whz, whn = (W_hh_gru[0:16].T, W_hh_gru[16:32].T, W_hh_gru[32:48].T)
    bir, biz, bin_ = r2(b_ih_gru[0:16]), r2(b_ih_gru[16:32]), r2(b_ih_gru[32:48])
    bhr, bhz, bhn = r2(b_hh_gru[0:16]), r2(b_hh_gru[16:32]), r2(b_hh_gru[32:48])
    gru_w = (r2(b_conv), wir, wiz, win, whr, whz, whn,
             bir, biz, bin_, bhr, bhz, bhn)

    bl = b_ih_lstm + b_hh_lstm
    lstm_w = []
    for g in range(4):
        rows = slice(16 * g, 16 * (g + 1))
        lstm_w += [W_ih_lstm[rows, 0:16].T, W_ih_lstm[rows, 16:32].T,
                   W_hh_lstm[rows].T, r2(bl[rows])]

    hidden = x
    xw = _xw_call(x, wbig)
    sc_call = _make_sc_call()
    for t in range(T_MP):
        aggp = sc_call(xw.reshape(N * NK, 16), gidx_p, dst_p)
        if t < T_MP - 1:
            hidden, xw = _gruxw_call(aggp[0], aggp[1], hidden, *gru_w, wbig)
        else:
            out = _final_call(aggp[0, :N], aggp[1, :N], hidden, *gru_w, *lstm_w,
                              W_p1[0:16], W_p1[16:32], r2(b_p1),
                              W_p2, r2(b_p2))
    return out


# speedup vs baseline: 18.7971x; 1.0882x over previous
import functools

import jax
import jax.numpy as jnp
from jax import lax
from jax.experimental import pallas as pl
from jax.experimental.pallas import tpu as pltpu
from jax.experimental.pallas import tpu_sc as plsc

N = 10000
E = 320000
D = 16
NK = 64
CHUNK = 128
EP = 327680
NCH = EP // CHUNK
NW = 32
CPW0 = 120
CPW1 = 40
NCH0 = 16 * CPW0
NSUB = 16
NP = 10112
RPS = NP // NSUB
T_MP = 3


def _pre_body(h_ref, eh0_ref, eh1_ref, eh2_ref, wp_ref, bp_ref,
              ee0_ref, ee1_ref, we1_ref, be1_ref, we2_ref, be2_ref,
              x_ref, ewt_ref):
    lanes = lax.broadcasted_iota(jnp.int32, (N, 16), 1)
    p0 = jnp.dot(eh0_ref[...], wp_ref[0:8, :], preferred_element_type=jnp.float32)
    p1 = jnp.dot(eh1_ref[...], wp_ref[8:12, :], preferred_element_type=jnp.float32)
    p2 = jnp.dot(eh2_ref[...], wp_ref[12:16, :], preferred_element_type=jnp.float32)
    oh0 = (h_ref[:, 0:1] == lanes).astype(jnp.float32)
    oh1 = (h_ref[:, 1:2] == lanes).astype(jnp.float32)
    oh2 = (h_ref[:, 2:3] == lanes).astype(jnp.float32)
    acc = jnp.dot(oh0, p0, preferred_element_type=jnp.float32)
    acc = acc + jnp.dot(oh1, p1, preferred_element_type=jnp.float32)
    acc = acc + jnp.dot(oh2, p2, preferred_element_type=jnp.float32)
    x_ref[...] = jnp.maximum(acc + bp_ref[...], 0.0)

    kcol = lax.broadcasted_iota(jnp.int32, (NK, 1), 0)
    lanes8 = lax.broadcasted_iota(jnp.int32, (NK, 8), 1)
    ohk0 = ((kcol // 8) == lanes8).astype(jnp.float32)
    ohk1 = ((kcol % 8) == lanes8).astype(jnp.float32)
    g0 = jnp.dot(ee0_ref[...], we1_ref[0:4, :], preferred_element_type=jnp.float32)
    g1 = jnp.dot(ee1_ref[...], we1_ref[4:8, :], preferred_element_type=jnp.float32)
    hmid = jnp.dot(ohk0, g0, preferred_element_type=jnp.float32)
    hmid = hmid + jnp.dot(ohk1, g1, preferred_element_type=jnp.float32)
    hmid = jnp.maximum(hmid + be1_ref[...], 0.0)
    ewt_ref[...] = jnp.dot(hmid, we2_ref[...], preferred_element_type=jnp.float32) + be2_ref[...]


_pre_call = pl.pallas_call(
    _pre_body,
    out_shape=(jax.ShapeDtypeStruct((N, 16), jnp.float32),
               jax.ShapeDtypeStruct((NK, 256), jnp.float32)),
)


_XW_BLK = 2000


def _xw_body(x_ref, wb_ref, o_ref):
    x = x_ref[...]
    for g in range(8):
        o_ref[g] = jnp.dot(x, wb_ref[g], preferred_element_type=jnp.float32)


_xw_call = pl.pallas_call(
    _xw_body,
    grid=(N // _XW_BLK,),
    in_specs=[pl.BlockSpec((_XW_BLK, 16), lambda i: (i, 0)),
              pl.BlockSpec((8, 16, 128), lambda i: (0, 0, 0))],
    out_specs=pl.BlockSpec((8, _XW_BLK, 128), lambda i: (0, i, 0)),
    out_shape=jax.ShapeDtypeStruct((8, N, 128), jnp.float32),
)


NBUF = 8
LOOK = NBUF - 2


def _sc_body(table, gidx, dstp, out, gidx_v, dst_v, rows_v, zbuf, agg_sh,
             sem_g, sem_s):
    c = lax.axis_index("c")
    s = lax.axis_index("s")

    def _z(i, carry):
        zbuf[i, :] = jnp.zeros((16,), jnp.float32)
        return carry
    lax.fori_loop(0, RPS, _z, 0)
    pltpu.sync_copy(zbuf, agg_sh.at[pl.ds(s * RPS, RPS)])

    @pl.when(c == 0)
    def _():
        pltpu.sync_copy(gidx.at[pl.ds(s * CPW0, CPW0)], gidx_v)
        pltpu.sync_copy(dstp.at[pl.ds(s * CPW0, CPW0)], dst_v)

    @pl.when(c == 1)
    def _():
        base = NCH0 + s * CPW1
        pltpu.sync_copy(gidx.at[pl.ds(base, CPW1)],
                        gidx_v.at[pl.ds(0, CPW1)])
        pltpu.sync_copy(dstp.at[pl.ds(base, CPW1)],
                        dst_v.at[pl.ds(0, CPW1)])
    cpw = jnp.where(c == 0, CPW0, CPW1)
    n_groups = jnp.where(c == 0, CPW0 // NBUF, CPW1 // NBUF)
    plsc.subcore_barrier()

    def _fire_g(j, b):
        pltpu.async_copy(table.at[gidx_v.at[j]], rows_v.at[b], sem_g.at[b])

    def _wait_g(j, b):
        pltpu.make_async_copy(table.at[gidx_v.at[j]], rows_v.at[b],
                              sem_g.at[b]).wait()

    def _fire_s(j, b):
        pltpu.async_copy(rows_v.at[b], agg_sh.at[dst_v.at[j]], sem_s.at[b],
                         add=True)

    def _wait_s(j, b):
        pltpu.make_async_copy(rows_v.at[b], agg_sh.at[dst_v.at[j]],
                              sem_s.at[b]).wait()

    def _step(j, b, do_wait_s, do_fire_g):
        bg = (b + LOOK) % NBUF
        _wait_g(j, b)
        _fire_s(j, b)
        if do_wait_s:
            _wait_s(j - 2, bg)
        if do_fire_g:
            _fire_g(j + LOOK, bg)

    for b in range(LOOK):
        _fire_g(b, b)
    for b in range(NBUF):
        _step(b, b, b >= 2, True)

    def _group(jj, carry):
        j0 = jj * NBUF
        for b in range(NBUF):
            _step(j0 + b, b, True, True)
        return carry
    lax.fori_loop(1, n_groups - 1, _group, 0)

    for b in range(NBUF):
        _step(cpw - NBUF + b, b, True, b < NBUF - LOOK)
    _wait_s(cpw - 2, NBUF - 2)
    _wait_s(cpw - 1, NBUF - 1)
    plsc.subcore_barrier()

    pltpu.sync_copy(agg_sh.at[pl.ds(s * RPS, RPS)],
                    out.at[c, pl.ds(s * RPS, RPS)])


def _make_sc_call():
  return pl.kernel(
    _sc_body,
    out_type=jax.ShapeDtypeStruct((2, NP, 16), jnp.float32),
    mesh=plsc.VectorSubcoreMesh(core_axis_name="c", subcore_axis_name="s",
                                num_cores=2, num_subcores=NSUB),
    scratch_types=[
        pltpu.VMEM((CPW0, CHUNK), jnp.int32),
        pltpu.VMEM((CPW0, CHUNK), jnp.int32),
        pltpu.VMEM((NBUF, CHUNK, 16), jnp.float32),
        pltpu.VMEM((RPS, 16), jnp.float32),
        pltpu.VMEM_SHARED((NP, 16), jnp.float32),
        pltpu.SemaphoreType.DMA((NBUF,)),
        pltpu.SemaphoreType.DMA((NBUF,)),
    ],
    compiler_params=pltpu.CompilerParams(use_tc_tiling_on_sc=False),
  )


def _gru_step(a0, a1, hid, bc, wir, wiz, win, whr, whz, whn,
              bir, biz, bin_, bhr, bhz, bhn):
    x_in = jnp.maximum(a0 + a1 + bc, 0.0)
    dot = functools.partial(jnp.dot, preferred_element_type=jnp.float32)
    r = jax.nn.sigmoid(dot(x_in, wir) + bir + dot(hid, whr) + bhr)
    z = jax.nn.sigmoid(dot(x_in, wiz) + biz + dot(hid, whz) + bhz)
    n = jnp.tanh(dot(x_in, win) + bin_ + r * (dot(hid, whn) + bhn))
    return (1.0 - z) * n + z * hid


def _gruxw_body(a0_ref, a1_ref, hid_ref, bc_ref, wir_ref, wiz_ref, win_ref,
                whr_ref, whz_ref, whn_ref, bir_ref, biz_ref, bin_ref,
                bhr_ref, bhz_ref, bhn_ref, wb_ref, hout_ref, xw_ref):
    hnew = _gru_step(a0_ref[...], a1_ref[...], hid_ref[...], bc_ref[...],
                     wir_ref[...], wiz_ref[...], win_ref[...],
                     whr_ref[...], whz_ref[...], whn_ref[...],
                     bir_ref[...], biz_ref[...], bin_ref[...],
                     bhr_ref[...], bhz_ref[...], bhn_ref[...])
    hout_ref[...] = hnew
    for g in range(8):
        xw_ref[g] = jnp.dot(hnew, wb_ref[g], preferred_element_type=jnp.float32)


_GRU_BLK = 2000
_w16 = pl.BlockSpec((16, 16), lambda i: (0, 0))
_b16 = pl.BlockSpec((1, 16), lambda i: (0, 0))

_gruxw_call = pl.pallas_call(
    _gruxw_body,
    grid=(N // _GRU_BLK,),
    in_specs=[pl.BlockSpec((_GRU_BLK, 16), lambda i: (i, 0)),
              pl.BlockSpec((_GRU_BLK, 16), lambda i: (i, 0)),
              pl.BlockSpec((_GRU_BLK, 16), lambda i: (i, 0)),
              _b16, _w16, _w16, _w16, _w16, _w16, _w16,
              _b16, _b16, _b16, _b16, _b16, _b16,
              pl.BlockSpec((8, 16, 128), lambda i: (0, 0, 0))],
    out_specs=(pl.BlockSpec((_GRU_BLK, 16), lambda i: (i, 0)),
               pl.BlockSpec((8, _GRU_BLK, 128), lambda i: (0, i, 0))),
    out_shape=(jax.ShapeDtypeStruct((N, 16), jnp.float32),
               jax.ShapeDtypeStruct((8, N, 128), jnp.float32)),
)


def _final_body(a0_ref, a1_ref, hid_ref, bc_ref, wir_ref, wiz_ref, win_ref,
                whr_ref, whz_ref, whn_ref, bir_ref, biz_ref, bin_ref,
                bhr_ref, bhz_ref, bhn_ref,
                aq_i_ref, ar_i_ref, hh_i_ref, bl_i_ref,
                aq_f_ref, ar_f_ref, hh_f_ref, bl_f_ref,
                aq_g_ref, ar_g_ref, hh_g_ref, bl_g_ref,
                aq_o_ref, ar_o_ref, hh_o_ref, bl_o_ref,
                wp1q_ref, wp1r_ref, bp1_ref, wp2_ref, bp2_ref, out_ref):
    x = _gru_step(a0_ref[...], a1_ref[...], hid_ref[...], bc_ref[...],
                  wir_ref[...], wiz_ref[...], win_ref[...],
                  whr_ref[...], whz_ref[...], whn_ref[...],
                  bir_ref[...], biz_ref[...], bin_ref[...],
                  bhr_ref[...], bhz_ref[...], bhn_ref[...])
    dot = functools.partial(jnp.dot, preferred_element_type=jnp.float32)
    hc = jnp.zeros((1, 16), jnp.float32)
    cc = jnp.zeros((1, 16), jnp.float32)
    q = jnp.zeros((1, 16), jnp.float32)
    readout = jnp.zeros((1, 16), jnp.float32)
    for _ in range(3):
        i_g = jax.nn.sigmoid(dot(q, aq_i_ref[...]) + dot(readout, ar_i_ref[...])
                             + dot(hc, hh_i_ref[...]) + bl_i_ref[...])
        f_g = jax.nn.sigmoid(dot(q, aq_f_ref[...]) + dot(readout, ar_f_ref[...])
                             + dot(hc, hh_f_ref[...]) + bl_f_ref[...])
        g_g = jnp.tanh(dot(q, aq_g_ref[...]) + dot(readout, ar_g_ref[...])
                       + dot(hc, hh_g_ref[...]) + bl_g_ref[...])
        o_g = jax.nn.sigmoid(dot(q, aq_o_ref[...]) + dot(readout, ar_o_ref[...])
                             + dot(hc, hh_o_ref[...]) + bl_o_ref[...])
        cc = f_g * cc + i_g * g_g
        hc = o_g * jnp.tanh(cc)
        q = hc
        en = jnp.sum(x * q, axis=1, keepdims=True)
        m = jnp.max(en, axis=0, keepdims=True)
        ex = jnp.exp(en - m)
        alpha = ex / jnp.sum(ex, axis=0, keepdims=True)
        readout = jnp.sum(x * alpha, axis=0, keepdims=True)
    hid1 = jnp.maximum(dot(q, wp1q_ref[...]) + dot(readout, wp1r_ref[...])
                       + bp1_ref[...], 0.0)
    out_ref[...] = dot(hid1, wp2_ref[...]) + bp2_ref[...]


_final_call = pl.pallas_call(
    _final_body,
    out_shape=jax.ShapeDtypeStruct((1, 16), jnp.float32),
)


def kernel(edge_index, h, e, Eh0, Eh1, Eh2, Ee0, Ee1, W_proj, b_proj,
           W_e1, b_e1, W_e2, b_e2, b_conv,
           W_ih_gru, W_hh_gru, b_ih_gru, b_hh_gru,
           W_ih_lstm, W_hh_lstm, b_ih_lstm, b_hh_lstm,
           W_p1, b_p1, W_p2, b_p2):
    src = edge_index[0].astype(jnp.int32)
    dst = edge_index[1].astype(jnp.int32)
    e0 = e[:, 0].astype(jnp.int32)
    e1 = e[:, 1].astype(jnp.int32)
    gidx = e1 * (N * 8) + src * 8 + e0
    gidx_p = jnp.pad(gidx, (0, EP - E)).reshape(NCH, CHUNK)
    dst_p = jnp.pad(dst, (0, EP - E), constant_values=N).reshape(NCH, CHUNK)

    r2 = lambda v: v.reshape(1, -1)
    x, ewt = _pre_call(h.astype(jnp.int32), Eh0, Eh1, Eh2, W_proj, r2(b_proj),
                       Ee0, Ee1, W_e1, r2(b_e1), W_e2, r2(b_e2))
    wbig = ewt.reshape(8, 8, 16, 16).transpose(1, 2, 0, 3).reshape(8, 16, 128)

    wir, wiz, win = (W_ih_gru[0:16].T, W_ih_gru[16:32].T, W_ih_gru[32:48].T)
    whr, whz, whn = (W_hh_gru[0:16].T, W_hh_gru[16:32].T, W_hh_gru[32:48].T)
    bir, biz, bin_ = r2(b_ih_gru[0:16]), r2(b_ih_gru[16:32]), r2(b_ih_gru[32:48])
    bhr, bhz, bhn = r2(b_hh_gru[0:16]), r2(b_hh_gru[16:32]), r2(b_hh_gru[32:48])
    gru_w = (r2(b_conv), wir, wiz, win, whr, whz, whn,
             bir, biz, bin_, bhr, bhz, bhn)

    bl = b_ih_lstm + b_hh_lstm
    lstm_w = []
    for g in range(4):
        rows = slice(16 * g, 16 * (g + 1))
        lstm_w += [W_ih_lstm[rows, 0:16].T, W_ih_lstm[rows, 16:32].T,
                   W_hh_lstm[rows].T, r2(bl[rows])]

    hidden = x
    xw = _xw_call(x, wbig)
    sc_call = _make_sc_call()
    for t in range(T_MP):
        aggp = sc_call(xw.reshape(N * NK, 16), gidx_p, dst_p)
        if t < T_MP - 1:
            hidden, xw = _gruxw_call(aggp[0], aggp[1], hidden, *gru_w, wbig)
        else:
            out = _final_call(aggp[0, :N], aggp[1, :N], hidden, *gru_w, *lstm_w,
                              W_p1[0:16], W_p1[16:32], r2(b_p1),
                              W_p2, r2(b_p2))
    return out
